# Initial kernel scaffold; baseline (speedup 1.0000x reference)
import jax
import jax.numpy as jnp
from jax.experimental import pallas as pl


def kernel(content, timestamp, context, query_content, query_context, idx, mem_content, mem_contexts, mem_strengths, mem_freshness, Wt1, bt1, Wt2, bt2, Wc1, bc1, Wc2, bc2, Ws1, bs1, Ws2, bs2):
    raise NotImplementedError("write your pallas kernel here")


# speedup vs baseline: 1.5510x; 1.5510x over previous
import functools

import jax
import jax.numpy as jnp
from jax import lax
from jax.experimental import pallas as pl
from jax.experimental.pallas import tpu as pltpu

CAP = 100000
CD = 128
CXD = 128
B = 1024
K = 8
TN = 2000
NT = CAP // TN


def _norm_rows(x):
    return x / (jnp.sqrt(jnp.sum(x * x, axis=-1, keepdims=True)) + 1e-8)


def _prep_body(ctx_ref, wc1_ref, bc1_ref, wc2_ref, bc2_ref, qc_ref, qx_ref,
               ctxe_out, qn_out, qcn_out):
    h = jnp.maximum(
        jnp.dot(ctx_ref[...], wc1_ref[...], preferred_element_type=jnp.float32)
        + bc1_ref[...], 0.0)
    ctxe_out[...] = (
        jnp.dot(h, wc2_ref[...], preferred_element_type=jnp.float32)
        + bc2_ref[...])
    qn_out[...] = _norm_rows(qc_ref[...])
    qcn_out[...] = _norm_rows(qx_ref[...])


def _prep(context, Wc1, bc1, Wc2, bc2, query_content, query_context):
    return pl.pallas_call(
        _prep_body,
        out_shape=[
            jax.ShapeDtypeStruct((B, CXD), jnp.float32),
            jax.ShapeDtypeStruct((B, CD), jnp.float32),
            jax.ShapeDtypeStruct((B, CXD), jnp.float32),
        ],
    )(context, Wc1, bc1.reshape(1, CXD), Wc2, bc2.reshape(1, CXD),
      query_content, query_context)


def _mega_body(qn_ref, qcn_ref, mc_ref, mx_ref, mf_ref,
               sim_out, idx_out, tw_out, rv_ref, ri_ref, rt_ref):
    t = pl.program_id(0)
    nt = pl.num_programs(0)

    @pl.when(t == 0)
    def _():
        rv_ref[...] = jnp.full((B, K), -jnp.inf, jnp.float32)
        ri_ref[...] = jnp.zeros((B, K), jnp.int32)
        rt_ref[...] = jnp.zeros((B, K), jnp.float32)

    mc = mc_ref[...]
    cn = _norm_rows(mc)
    cs = lax.dot_general(qn_ref[...], cn, (((1,), (1,)), ((), ())),
                         preferred_element_type=jnp.float32)
    mx = mx_ref[...]
    xn = _norm_rows(mx)
    xs = lax.dot_general(qcn_ref[...], xn, (((1,), (1,)), ((), ())),
                         preferred_element_type=jnp.float32)
    mf = mf_ref[0]
    f = 0.5 * cs + 0.3 * xs + 0.2 * mf
    col = t * TN + lax.broadcasted_iota(jnp.int32, (B, TN), 1)

    cand_v = jnp.concatenate([rv_ref[...], f], axis=1)
    cand_i = jnp.concatenate([ri_ref[...], col], axis=1)
    cand_t = jnp.concatenate([rt_ref[...], jnp.broadcast_to(mf, (B, TN))],
                             axis=1)
    vs, ids, tws = [], [], []
    for _ in range(K):
        m = jnp.max(cand_v, axis=1, keepdims=True)
        eq = cand_v == m
        si = jnp.min(jnp.where(eq, cand_i, jnp.int32(2**30)), axis=1,
                     keepdims=True)
        oh = eq & (cand_i == si)
        tw = jnp.sum(jnp.where(oh, cand_t, 0.0), axis=1, keepdims=True)
        vs.append(m)
        ids.append(si)
        tws.append(tw)
        cand_v = jnp.where(oh, -jnp.inf, cand_v)
    rv_ref[...] = jnp.concatenate(vs, axis=1)
    ri_ref[...] = jnp.concatenate(ids, axis=1)
    rt_ref[...] = jnp.concatenate(tws, axis=1)

    @pl.when(t == nt - 1)
    def _():
        sim_out[...] = rv_ref[...]
        idx_out[...] = ri_ref[...]
        tw_out[...] = rt_ref[...]


def _mega(qn, qcn, mc, mx, mf):
    full = lambda shape: pl.BlockSpec(shape, lambda t: (0,) * len(shape))
    return pl.pallas_call(
        _mega_body,
        grid=(NT,),
        in_specs=[
            full((B, CD)),
            full((B, CXD)),
            pl.BlockSpec((TN, CD), lambda t: (t, 0)),
            pl.BlockSpec((TN, CXD), lambda t: (t, 0)),
            pl.BlockSpec((1, 1, TN), lambda t: (t, 0, 0)),
        ],
        out_specs=[
            pl.BlockSpec((B, K), lambda t: (0, 0)),
            pl.BlockSpec((B, K), lambda t: (0, 0)),
            pl.BlockSpec((B, K), lambda t: (0, 0)),
        ],
        out_shape=[
            jax.ShapeDtypeStruct((B, K), jnp.float32),
            jax.ShapeDtypeStruct((B, K), jnp.int32),
            jax.ShapeDtypeStruct((B, K), jnp.float32),
        ],
        scratch_shapes=[
            pltpu.VMEM((B, K), jnp.float32),
            pltpu.VMEM((B, K), jnp.int32),
            pltpu.VMEM((B, K), jnp.float32),
        ],
        compiler_params=pltpu.CompilerParams(
            dimension_semantics=("arbitrary",)),
    )(qn, qcn, mc, mx, mf.reshape(NT, 1, TN))


def kernel(content, timestamp, context, query_content, query_context, idx,
           mem_content, mem_contexts, mem_strengths, mem_freshness,
           Wt1, bt1, Wt2, bt2, Wc1, bc1, Wc2, bc2, Ws1, bs1, Ws2, bs2):
    ctx_enc, qn, qcn = _prep(context, Wc1, bc1, Wc2, bc2,
                             query_content, query_context)
    mc = mem_content.at[idx].set(content)
    mx = mem_contexts.at[idx].set(ctx_enc)
    mf = mem_freshness.at[idx].set(1.0)
    top_sim, top_idx, top_tw = _mega(qn, qcn, mc, mx, mf)
    retrieved = mc[top_idx]
    return retrieved, top_sim, top_tw
